# Initial kernel scaffold; baseline (speedup 1.0000x reference)
import jax
import jax.numpy as jnp
from jax.experimental import pallas as pl


def kernel(x, sequence_lengths, indices):
    raise NotImplementedError("write your pallas kernel here")


# speedup vs baseline: 6.8996x; 6.8996x over previous
import functools

import jax
import jax.numpy as jnp
from jax import lax
from jax.experimental import pallas as pl
from jax.experimental.pallas import tpu as pltpu
from jax.experimental.pallas import tpu_sc as plsc

POOL = 2
KEYSH = 11


def _flip(v, m, lane):
    n = v.shape[1]
    lo = (lane & m) == 0
    return jnp.where(lo, pltpu.roll(v, n - m, axis=1), pltpu.roll(v, m, axis=1))


def _sort_body(v_ref, out_ref):
    v = v_ref[...]
    rb, n = v.shape
    lane = lax.broadcasted_iota(jnp.int32, (rb, n), 1)
    size = 2
    while size <= n:
        p = v
        b = 1
        while b < size:
            p = _flip(p, b, lane)
            b <<= 1
        top = size >> 1
        lo = (lane & top) == 0
        kv = v >> KEYSH
        kp = p >> KEYSH
        swap = (lo & (kv > kp)) | (jnp.logical_not(lo) & (kp > kv))
        v = jnp.where(swap, p, v)
        st = size >> 2
        while st >= 1:
            p = _flip(v, st, lane)
            lo = (lane & st) == 0
            kv = v >> KEYSH
            kp = p >> KEYSH
            swap = (lo & (kv > kp)) | (jnp.logical_not(lo) & (kp > kv))
            v = jnp.where(swap, p, v)
            st >>= 1
        size <<= 1
    out_ref[...] = v


@functools.lru_cache(maxsize=None)
def _build_sorter(R, L, interpret=False):
    RB = 8
    return pl.pallas_call(
        _sort_body,
        grid=(R // RB,),
        in_specs=[pl.BlockSpec((RB, L), lambda i: (i, 0))],
        out_specs=pl.BlockSpec((RB, L), lambda i: (i, 0)),
        out_shape=jax.ShapeDtypeStruct((R, L), jnp.int32),
        interpret=interpret,
    )


@functools.lru_cache(maxsize=None)
def _build_scatter(B, C, L):
    R = B * C
    OUT_L = L * POOL
    info = plsc.get_sparse_core_info()
    NC, NS, NL = info.num_cores, info.num_subcores, info.num_lanes
    NW = NC * NS
    assert R % NW == 0 and L % NL == 0
    rows_per_w = R // NW

    mesh = plsc.VectorSubcoreMesh(core_axis_name="c", subcore_axis_name="s")

    @functools.partial(
        pl.kernel,
        mesh=mesh,
        compiler_params=pltpu.CompilerParams(needs_layout_passes=False),
        out_type=(
            jax.ShapeDtypeStruct((R, OUT_L), jnp.float32),
            jax.ShapeDtypeStruct((B,), jnp.int32),
        ),
        scratch_types=[
            pltpu.VMEM((L,), jnp.float32),
            pltpu.VMEM((L + 16,), jnp.int32),
            pltpu.VMEM((OUT_L,), jnp.float32),
            pltpu.VMEM((B,), jnp.int32),
        ],
    )
    def k(x_hbm, seq_hbm, sv_hbm, out_hbm, seqout_hbm, x_v, sv_v, out_v, seq_v):
        cid = lax.axis_index("c")
        sid = lax.axis_index("s")
        wid = sid * NC + cid
        base = wid * rows_per_w

        @pl.when(wid == 0)
        def _():
            pltpu.sync_copy(seq_hbm, seq_v)
            for j in range(B // NL):
                v = seq_v[pl.ds(j * NL, NL)]
                seq_v[pl.ds(j * NL, NL)] = jnp.maximum(v * POOL, OUT_L)
            pltpu.sync_copy(seq_v, seqout_hbm)

        neg1 = jnp.full((NL,), -1, jnp.int32)
        sv_v[pl.ds(L, NL)] = neg1

        def body(r, carry):
            row = base + r
            pltpu.sync_copy(x_hbm.at[row], x_v)
            pltpu.sync_copy(sv_hbm.at[row], sv_v.at[pl.ds(0, L)])
            zeros = jnp.zeros((NL,), jnp.float32)
            for j in range(OUT_L // NL):
                out_v[pl.ds(j * NL, NL)] = zeros
            for j in range(L // NL):
                sv = sv_v[pl.ds(j * NL, NL)]
                nxt = sv_v[pl.ds(j * NL + 1, NL)]
                slot = lax.shift_right_logical(sv, KEYSH)
                mask = slot != lax.shift_right_logical(nxt, KEYSH)
                pos = sv & (L - 1)
                val = plsc.load_gather(x_v, [pos])
                plsc.store_scatter(out_v, [slot], val, mask=mask)
            pltpu.sync_copy(out_v, out_hbm.at[row])
            return carry

        lax.fori_loop(0, rows_per_w, body, 0)

    return k


@jax.jit
def kernel(x, sequence_lengths, indices):
    B, C, L = x.shape
    R = B * C
    idx2 = indices.reshape(R, L)
    packed = (idx2 << KEYSH) | jax.lax.broadcasted_iota(jnp.int32, (R, L), 1)
    sorted_packed = _build_sorter(R, L)(packed)
    out_flat, seq_out = _build_scatter(B, C, L)(
        x.reshape(R, L), sequence_lengths, sorted_packed
    )
    return out_flat.reshape(B, C, L * POOL), seq_out


# speedup vs baseline: 10.5895x; 1.5348x over previous
import functools

import jax
import jax.numpy as jnp
from jax import lax
from jax.experimental import pallas as pl
from jax.experimental.pallas import tpu as pltpu
from jax.experimental.pallas import tpu_sc as plsc

POOL = 2
KEYSH = 11


def _flip(v, m, lane):
    n = v.shape[1]
    lo = (lane & m) == 0
    return jnp.where(lo, pltpu.roll(v, n - m, axis=1), pltpu.roll(v, m, axis=1))


def _sort_body(v_ref, out_ref):
    v = v_ref[...]
    rb, n = v.shape
    lane = lax.broadcasted_iota(jnp.int32, (rb, n), 1)
    size = 2
    while size <= n:
        p = v
        b = 1
        while b < size:
            p = _flip(p, b, lane)
            b <<= 1
        top = size >> 1
        lo = (lane & top) == 0
        kv = v >> KEYSH
        kp = p >> KEYSH
        swap = (lo & (kv > kp)) | (jnp.logical_not(lo) & (kp > kv))
        v = jnp.where(swap, p, v)
        st = size >> 2
        while st >= 1:
            p = _flip(v, st, lane)
            lo = (lane & st) == 0
            kv = v >> KEYSH
            kp = p >> KEYSH
            swap = (lo & (kv > kp)) | (jnp.logical_not(lo) & (kp > kv))
            v = jnp.where(swap, p, v)
            st >>= 1
        size <<= 1
    out_ref[...] = v


@functools.lru_cache(maxsize=None)
def _build_sorter(R, L, interpret=False):
    RB = 64
    return pl.pallas_call(
        _sort_body,
        grid=(R // RB,),
        in_specs=[pl.BlockSpec((RB, L), lambda i: (i, 0))],
        out_specs=pl.BlockSpec((RB, L), lambda i: (i, 0)),
        out_shape=jax.ShapeDtypeStruct((R, L), jnp.int32),
        interpret=interpret,
    )


@functools.lru_cache(maxsize=None)
def _build_scatter(B, C, L):
    R = B * C
    OUT_L = L * POOL
    info = plsc.get_sparse_core_info()
    NC, NS, NL = info.num_cores, info.num_subcores, info.num_lanes
    NW = NC * NS
    assert R % NW == 0 and L % NL == 0
    rows_per_w = R // NW

    mesh = plsc.VectorSubcoreMesh(core_axis_name="c", subcore_axis_name="s")

    @functools.partial(
        pl.kernel,
        mesh=mesh,
        compiler_params=pltpu.CompilerParams(needs_layout_passes=False),
        out_type=(
            jax.ShapeDtypeStruct((R, OUT_L), jnp.float32),
            jax.ShapeDtypeStruct((B,), jnp.int32),
        ),
        scratch_types=[
            pltpu.VMEM((L,), jnp.float32),
            pltpu.VMEM((L + 16,), jnp.int32),
            pltpu.VMEM((OUT_L,), jnp.float32),
            pltpu.VMEM((B,), jnp.int32),
        ],
    )
    def k(x_hbm, seq_hbm, sv_hbm, out_hbm, seqout_hbm, x_v, sv_v, out_v, seq_v):
        cid = lax.axis_index("c")
        sid = lax.axis_index("s")
        wid = sid * NC + cid
        base = wid * rows_per_w

        @pl.when(wid == 0)
        def _():
            pltpu.sync_copy(seq_hbm, seq_v)
            for j in range(B // NL):
                v = seq_v[pl.ds(j * NL, NL)]
                seq_v[pl.ds(j * NL, NL)] = jnp.maximum(v * POOL, OUT_L)
            pltpu.sync_copy(seq_v, seqout_hbm)

        neg1 = jnp.full((NL,), -1, jnp.int32)
        sv_v[pl.ds(L, NL)] = neg1

        def body(r, carry):
            row = base + r
            pltpu.sync_copy(x_hbm.at[row], x_v)
            pltpu.sync_copy(sv_hbm.at[row], sv_v.at[pl.ds(0, L)])
            zeros = jnp.zeros((NL,), jnp.float32)
            for j in range(OUT_L // NL):
                out_v[pl.ds(j * NL, NL)] = zeros
            for j in range(L // NL):
                sv = sv_v[pl.ds(j * NL, NL)]
                nxt = sv_v[pl.ds(j * NL + 1, NL)]
                slot = lax.shift_right_logical(sv, KEYSH)
                mask = slot != lax.shift_right_logical(nxt, KEYSH)
                pos = sv & (L - 1)
                val = plsc.load_gather(x_v, [pos])
                plsc.store_scatter(out_v, [slot], val, mask=mask)
            pltpu.sync_copy(out_v, out_hbm.at[row])
            return carry

        lax.fori_loop(0, rows_per_w, body, 0)

    return k


@jax.jit
def kernel(x, sequence_lengths, indices):
    B, C, L = x.shape
    R = B * C
    idx2 = indices.reshape(R, L)
    packed = (idx2 << KEYSH) | jax.lax.broadcasted_iota(jnp.int32, (R, L), 1)
    sorted_packed = _build_sorter(R, L)(packed)
    out_flat, seq_out = _build_scatter(B, C, L)(
        x.reshape(R, L), sequence_lengths, sorted_packed
    )
    return out_flat.reshape(B, C, L * POOL), seq_out


# speedup vs baseline: 12.7399x; 1.2031x over previous
import functools

import jax
import jax.numpy as jnp
from jax import lax
from jax.experimental import pallas as pl
from jax.experimental.pallas import tpu as pltpu
from jax.experimental.pallas import tpu_sc as plsc

POOL = 2
KEYSH = 11


def _flip(v, m, lane):
    n = v.shape[1]
    lo = (lane & m) == 0
    return jnp.where(lo, pltpu.roll(v, n - m, axis=1), pltpu.roll(v, m, axis=1))


def _sort_body(v_ref, out_ref):
    idx = v_ref[...]
    rb, n = idx.shape
    lane = lax.broadcasted_iota(jnp.int32, (rb, n), 1)
    v = (idx << KEYSH) | lane
    size = 2
    while size <= n:
        p = v
        b = 1
        while b < size:
            p = _flip(p, b, lane)
            b <<= 1
        top = size >> 1
        lo = (lane & top) == 0
        kv = v >> KEYSH
        kp = p >> KEYSH
        swap = (lo & (kv > kp)) | (jnp.logical_not(lo) & (kp > kv))
        v = jnp.where(swap, p, v)
        st = size >> 2
        while st >= 1:
            p = _flip(v, st, lane)
            lo = (lane & st) == 0
            kv = v >> KEYSH
            kp = p >> KEYSH
            swap = (lo & (kv > kp)) | (jnp.logical_not(lo) & (kp > kv))
            v = jnp.where(swap, p, v)
            st >>= 1
        size <<= 1
    out_ref[...] = v


@functools.lru_cache(maxsize=None)
def _build_sorter(R, L, interpret=False):
    RB = 128
    return pl.pallas_call(
        _sort_body,
        grid=(R // RB,),
        in_specs=[pl.BlockSpec((RB, L), lambda i: (i, 0))],
        out_specs=pl.BlockSpec((RB, L), lambda i: (i, 0)),
        out_shape=jax.ShapeDtypeStruct((R, L), jnp.int32),
        interpret=interpret,
    )


@functools.lru_cache(maxsize=None)
def _build_scatter(B, C, L):
    R = B * C
    OUT_L = L * POOL
    info = plsc.get_sparse_core_info()
    NC, NS, NL = info.num_cores, info.num_subcores, info.num_lanes
    NW = NC * NS
    assert R % NW == 0 and L % NL == 0
    rows_per_w = R // NW

    mesh = plsc.VectorSubcoreMesh(core_axis_name="c", subcore_axis_name="s")

    @functools.partial(
        pl.kernel,
        mesh=mesh,
        compiler_params=pltpu.CompilerParams(needs_layout_passes=False),
        out_type=(
            jax.ShapeDtypeStruct((R, OUT_L), jnp.float32),
            jax.ShapeDtypeStruct((B,), jnp.int32),
        ),
        scratch_types=[
            pltpu.VMEM((L,), jnp.float32),
            pltpu.VMEM((L + 16,), jnp.int32),
            pltpu.VMEM((OUT_L,), jnp.float32),
            pltpu.VMEM((B,), jnp.int32),
        ],
    )
    def k(x_hbm, seq_hbm, sv_hbm, out_hbm, seqout_hbm, x_v, sv_v, out_v, seq_v):
        cid = lax.axis_index("c")
        sid = lax.axis_index("s")
        wid = sid * NC + cid
        base = wid * rows_per_w

        @pl.when(wid == 0)
        def _():
            pltpu.sync_copy(seq_hbm, seq_v)
            for j in range(B // NL):
                v = seq_v[pl.ds(j * NL, NL)]
                seq_v[pl.ds(j * NL, NL)] = jnp.maximum(v * POOL, OUT_L)
            pltpu.sync_copy(seq_v, seqout_hbm)

        neg1 = jnp.full((NL,), -1, jnp.int32)
        sv_v[pl.ds(L, NL)] = neg1

        def body(r, carry):
            row = base + r
            pltpu.sync_copy(x_hbm.at[row], x_v)
            pltpu.sync_copy(sv_hbm.at[row], sv_v.at[pl.ds(0, L)])
            zeros = jnp.zeros((NL,), jnp.float32)
            for j in range(OUT_L // NL):
                out_v[pl.ds(j * NL, NL)] = zeros
            for j in range(L // NL):
                sv = sv_v[pl.ds(j * NL, NL)]
                nxt = sv_v[pl.ds(j * NL + 1, NL)]
                slot = lax.shift_right_logical(sv, KEYSH)
                mask = slot != lax.shift_right_logical(nxt, KEYSH)
                pos = sv & (L - 1)
                val = plsc.load_gather(x_v, [pos])
                plsc.store_scatter(out_v, [slot], val, mask=mask)
            pltpu.sync_copy(out_v, out_hbm.at[row])
            return carry

        lax.fori_loop(0, rows_per_w, body, 0)

    return k


@jax.jit
def kernel(x, sequence_lengths, indices):
    B, C, L = x.shape
    R = B * C
    sorted_packed = _build_sorter(R, L)(indices.reshape(R, L))
    out_flat, seq_out = _build_scatter(B, C, L)(
        x.reshape(R, L), sequence_lengths, sorted_packed
    )
    return out_flat.reshape(B, C, L * POOL), seq_out
